# Initial kernel scaffold; baseline (speedup 1.0000x reference)
import jax
import jax.numpy as jnp
from jax.experimental import pallas as pl


def kernel(z_, W, b, emb):
    raise NotImplementedError("write your pallas kernel here")


# speedup vs baseline: 1.0244x; 1.0244x over previous
import jax
import jax.numpy as jnp
from jax.experimental import pallas as pl

N_E = 8192
E_DIM = 64
NROWS = 8192
TM = 128
NT = NROWS // TM


def _conv_kernel(zt_ref, w_ref, o_ref):
    o_ref[...] = jax.lax.dot_general(
        zt_ref[...], w_ref[...], (((1,), (1,)), ((), ())),
        preferred_element_type=jnp.float32)


_conv_call = pl.pallas_call(
    _conv_kernel,
    grid=(8,),
    in_specs=[
        pl.BlockSpec((1024, 384), lambda i: (i, 0)),
        pl.BlockSpec((64, 384), lambda i: (0, 0)),
    ],
    out_specs=pl.BlockSpec((1024, 64), lambda i: (i, 0)),
    out_shape=jax.ShapeDtypeStruct((NROWS, 64), jnp.float32),
)


def _vq_kernel(zf_ref, emb_ref, idx_ref, dmin_ref, zq_ref, mark_ref, s_ref):
    i = pl.program_id(0)
    zft = zf_ref[...]
    e = emb_ref[...]
    cn = jnp.sum(e * e, axis=1)
    mm = jax.lax.dot_general(zft, e, (((1,), (1,)), ((), ())),
                             preferred_element_type=jnp.float32)
    rn = jnp.sum(zft * zft, axis=1, keepdims=True)
    dmat = (rn + cn[None, :]) - 2.0 * mm
    minval = jnp.min(dmat, axis=1, keepdims=True)
    iota = jax.lax.broadcasted_iota(jnp.int32, dmat.shape, 1)
    big = jnp.full(dmat.shape, 2 ** 30, jnp.int32)
    lidx = jnp.min(jnp.where(dmat == minval, iota, big), axis=1)
    idx_ref[0, 0, :] = lidx
    dmin_ref[0, 0, :] = minval[:, 0]
    oh = (iota == lidx[:, None]).astype(jnp.float32)
    zq_ref[0] = jax.lax.dot_general(oh, e, (((1,), (0,)), ((), ())),
                                    preferred_element_type=jnp.float32,
                                    precision=jax.lax.Precision.HIGHEST)

    @pl.when(i == 0)
    def _():
        mark_ref[...] = jnp.zeros_like(mark_ref)
        nrm = jnp.sqrt(jnp.sum(e * e, axis=1, keepdims=True))
        s_ref[0, 0, :] = jnp.sum(e / nrm, axis=0)

    mark_ref[0, 0, :] = jnp.maximum(mark_ref[0, 0, :], jnp.max(oh, axis=0))


_vq_call = pl.pallas_call(
    _vq_kernel,
    grid=(NT,),
    in_specs=[
        pl.BlockSpec((TM, 64), lambda i: (i, 0)),
        pl.BlockSpec((N_E, 64), lambda i: (0, 0)),
    ],
    out_specs=[
        pl.BlockSpec((1, 1, TM), lambda i: (i, 0, 0)),
        pl.BlockSpec((1, 1, TM), lambda i: (i, 0, 0)),
        pl.BlockSpec((1, TM, 64), lambda i: (i, 0, 0)),
        pl.BlockSpec((1, 1, N_E), lambda i: (0, 0, 0)),
        pl.BlockSpec((1, 1, 64), lambda i: (0, 0, 0)),
    ],
    out_shape=[
        jax.ShapeDtypeStruct((NT, 1, TM), jnp.int32),
        jax.ShapeDtypeStruct((NT, 1, TM), jnp.float32),
        jax.ShapeDtypeStruct((NT, TM, 64), jnp.float32),
        jax.ShapeDtypeStruct((1, 1, N_E), jnp.float32),
        jax.ShapeDtypeStruct((1, 1, 64), jnp.float32),
    ],
)


def kernel(z_, W, b, emb):
    B, C, H, Wd = z_.shape
    zt = z_.transpose(0, 2, 3, 1).reshape(B * H * Wd, C)
    zp = _conv_call(zt, W) + b[None, :]
    zf = (zp.reshape(B, H // 2, 2, Wd, E_DIM)
            .transpose(0, 4, 1, 2, 3)
            .reshape(NROWS, E_DIM))
    idx3, dmin3, zq3, mark, s = _vq_call(zf, emb)
    idx = idx3.reshape(B, -1)
    z_q = zq3.reshape(B, E_DIM, H, Wd)
    loss = (jnp.sum(dmin3) * ((1.0 + 0.25) / (NROWS * E_DIM))
            + jnp.sum(s * s) / (N_E * N_E))
    sampled = jnp.zeros((B, N_E), jnp.float32).at[0, :].set(mark[0, 0, :])
    return z_q, loss, sampled, idx
